# baseline (speedup 1.0000x reference)
import jax
import jax.numpy as jnp
from jax import lax
from jax.experimental import pallas as pl
from jax.experimental.pallas import tpu as pltpu
from jax.experimental.pallas import tpu_sc as plsc

B, L, D = 4096, 200, 64
VOCAB = 1000000
NUM_CLASSES = 1000
D2 = D // 2
H4 = 256000
FLAT_MINUS = 4 * H4 - 1

NC, NS = 2, 16
NW = NC * NS
ROWS_W = B // NW
TOK_W = ROWS_W * L
SUB = 128
KSUB = 4
CHUNK = SUB * KSUB
NCHUNK = TOK_W // CHUNK
XROWS_W = TOK_W // SUB


def _sc_pool_body(x_hbm, tbl_hbm, feat_hbm,
                  idx0, idx1, rows0, rows1, seg2d, acc_sh, sem0, sem1):
  c = lax.axis_index("c")
  s = lax.axis_index("s")
  w = c * NS + s
  xrow0 = w * XROWS_W
  sbase = s * ROWS_W

  iota = lax.iota(jnp.int32, 16)
  lvec = jnp.full((16,), L, jnp.int32)

  def seg_step(v, carry):
    t = iota + v * 16
    seg = lax.div(t, lvec) + sbase
    seg2d[v // 8, pl.ds((v % 8) * 16, 16)] = seg
    return carry
  lax.fori_loop(0, TOK_W // 16, seg_step, 0)

  zero16 = jnp.zeros((16,), jnp.float32)
  def z_step(i, carry):
    rows0[i // 2, pl.ds((i % 2) * 16, 16)] = zero16
    return carry
  lax.fori_loop(0, ROWS_W * (D2 // 16), z_step, 0)
  pltpu.sync_copy(rows0.at[pl.ds(0, ROWS_W)], acc_sh.at[pl.ds(sbase, ROWS_W)])

  bufs = ((idx0, rows0, sem0), (idx1, rows1, sem1))

  def load_and_fire(chunk, idx_ref, rows_ref, sem):
    pltpu.sync_copy(x_hbm.at[pl.ds(xrow0 + chunk * KSUB, KSUB)], idx_ref)
    for rj in range(KSUB):
      for k in range(SUB // 16):
        v = idx_ref[rj, pl.ds(k * 16, 16)]
        v4 = v + v + v + v
        idx_ref[rj, pl.ds(k * 16, 16)] = jnp.where(
            v >= 3 * H4, v4 - 3 * FLAT_MINUS,
            jnp.where(v >= 2 * H4, v4 - 2 * FLAT_MINUS,
                      jnp.where(v >= H4, v4 - FLAT_MINUS, v4)))
    for j in range(KSUB):
      pltpu.async_copy(tbl_hbm.at[idx_ref.at[j]],
                       rows_ref.at[pl.ds(j * SUB, SUB)], sem)

  load_and_fire(0, *bufs[0])
  load_and_fire(1, *bufs[1])

  def pair_body(g, carry):
    for bi in range(2):
      idx_ref, rows_ref, sem = bufs[bi]
      chunk = g * 2 + bi
      pltpu.make_async_copy(tbl_hbm.at[pl.ds(0, CHUNK)], rows_ref, sem).wait()
      for j in range(KSUB):
        pltpu.sync_copy(rows_ref.at[pl.ds(j * SUB, SUB)],
                        acc_sh.at[seg2d.at[chunk * KSUB + j]], add=True)
      nxt = chunk + 2
      @pl.when(nxt < NCHUNK)
      def _():
        load_and_fire(nxt, idx_ref, rows_ref, sem)
    return carry
  lax.fori_loop(0, NCHUNK // 2, pair_body, 0)

  pltpu.sync_copy(acc_sh.at[pl.ds(sbase, ROWS_W)],
                  feat_hbm.at[pl.ds(w * ROWS_W, ROWS_W)])


_sc_pool = pl.kernel(
    _sc_pool_body,
    out_type=jax.ShapeDtypeStruct((B, D2), jnp.float32),
    mesh=plsc.VectorSubcoreMesh(core_axis_name="c", subcore_axis_name="s"),
    scratch_types=[
        pltpu.VMEM((KSUB, SUB), jnp.int32),
        pltpu.VMEM((KSUB, SUB), jnp.int32),
        pltpu.VMEM((CHUNK, D2), jnp.float32),
        pltpu.VMEM((CHUNK, D2), jnp.float32),
        pltpu.VMEM((NCHUNK * KSUB, SUB), jnp.int32),
        pltpu.VMEM_SHARED((NS * ROWS_W, D2), jnp.float32),
        pltpu.SemaphoreType.DMA,
        pltpu.SemaphoreType.DMA,
    ],
    name="sc_embedding_bag_pool",
    compiler_params=pltpu.CompilerParams(use_tc_tiling_on_sc=False),
)

V_BLK = 2048
CONV_GRID = H4 // V_BLK
CLAMP_BLK = VOCAB // V_BLK


def _conv_body(r0_ref, r1_ref, r2_ref, r3_ref, out_ref):
  eye = (lax.broadcasted_iota(jnp.int32, (4 * D2, 4 * D2), 0)
         == lax.broadcasted_iota(jnp.int32, (4 * D2, 4 * D2), 1)
         ).astype(jnp.float32)
  both = jnp.concatenate(
      [r0_ref[...], r1_ref[...], r2_ref[...], r3_ref[...]], axis=0)
  out_ref[...] = lax.dot_general(both, eye, (((0,), (0,)), ((), ())),
                                 preferred_element_type=jnp.float32)


def _make_conv(rb):
  def mk(off):
    return lambda i: (rb, jnp.minimum(i + off, CLAMP_BLK))
  return pl.pallas_call(
      _conv_body,
      grid=(CONV_GRID,),
      in_specs=[pl.BlockSpec((D2, V_BLK), mk(o))
                for o in (0, CONV_GRID, 2 * CONV_GRID, 3 * CONV_GRID)],
      out_specs=pl.BlockSpec((V_BLK, 4 * D2), lambda i: (i, 0)),
      out_shape=jax.ShapeDtypeStruct((H4, 4 * D2), jnp.float32),
  )


_convA = _make_conv(0)
_convB = _make_conv(1)

NPAD = 1024
BLK_B = 512


def _mm_body(fa_ref, fb_ref, w_ref, b_ref, o_ref):
  f = jnp.concatenate([fa_ref[...], fb_ref[...]], axis=1)
  o_ref[...] = (
      jnp.dot(f, w_ref[...], preferred_element_type=jnp.float32,
              precision=lax.Precision.HIGHEST) * jnp.float32(1.0 / L)
      + b_ref[...])


_mm = pl.pallas_call(
    _mm_body,
    grid=(B // BLK_B,),
    in_specs=[
        pl.BlockSpec((BLK_B, D2), lambda i: (i, 0)),
        pl.BlockSpec((BLK_B, D2), lambda i: (i, 0)),
        pl.BlockSpec((D, NPAD), lambda i: (0, 0)),
        pl.BlockSpec((1, NPAD), lambda i: (0, 0)),
    ],
    out_specs=pl.BlockSpec((BLK_B, NPAD), lambda i: (i, 0)),
    out_shape=jax.ShapeDtypeStruct((B, NPAD), jnp.float32),
)


def kernel(x, table, W, b):
  x2d = x.astype(jnp.int32).reshape(-1, SUB)
  tt = table.T
  tA = _convA(tt, tt, tt, tt).reshape(4 * H4, D2)
  fA = _sc_pool(x2d, tA)
  tB = _convB(tt, tt, tt, tt).reshape(4 * H4, D2)
  fB = _sc_pool(x2d, tB)
  Wp = jnp.pad(W, ((0, 0), (0, NPAD - NUM_CLASSES)))
  bp = jnp.pad(b, (0, NPAD - NUM_CLASSES)).reshape(1, NPAD)
  out = _mm(fA, fB, Wp, bp)
  return out[:, :NUM_CLASSES]


# speedup vs baseline: 1.0585x; 1.0585x over previous
import jax
import jax.numpy as jnp
from jax import lax
from jax.experimental import pallas as pl
from jax.experimental.pallas import tpu as pltpu
from jax.experimental.pallas import tpu_sc as plsc

B, L, D = 4096, 200, 64
VOCAB = 1000000
NUM_CLASSES = 1000
D2 = D // 2
H4 = 262144
FLAT_MINUS = 4 * H4 - 1

NC, NS = 2, 16
NW = NC * NS
ROWS_W = B // NW
TOK_W = ROWS_W * L
SUB = 128
KSUB = 4
CHUNK = SUB * KSUB
NCHUNK = TOK_W // CHUNK
XROWS_W = TOK_W // SUB


def _sc_pool_body(x_hbm, tbl_hbm, feat_hbm,
                  idx0, idx1, rows0, rows1, seg2d, acc_sh, sem0, sem1):
  c = lax.axis_index("c")
  s = lax.axis_index("s")
  w = c * NS + s
  xrow0 = w * XROWS_W
  sbase = s * ROWS_W

  iota = lax.iota(jnp.int32, 16)
  lvec = jnp.full((16,), L, jnp.int32)

  def seg_step(v, carry):
    t = iota + v * 16
    seg = lax.div(t, lvec) + sbase
    seg2d[v // 8, pl.ds((v % 8) * 16, 16)] = seg
    return carry
  lax.fori_loop(0, TOK_W // 16, seg_step, 0)

  zero16 = jnp.zeros((16,), jnp.float32)
  def z_step(i, carry):
    rows0[i // 2, pl.ds((i % 2) * 16, 16)] = zero16
    return carry
  lax.fori_loop(0, ROWS_W * (D2 // 16), z_step, 0)
  pltpu.sync_copy(rows0.at[pl.ds(0, ROWS_W)], acc_sh.at[pl.ds(sbase, ROWS_W)])

  bufs = ((idx0, rows0, sem0), (idx1, rows1, sem1))

  def load_and_fire(chunk, idx_ref, rows_ref, sem):
    pltpu.sync_copy(x_hbm.at[pl.ds(xrow0 + chunk * KSUB, KSUB)], idx_ref)
    for rj in range(KSUB):
      for k in range(SUB // 16):
        v = idx_ref[rj, pl.ds(k * 16, 16)]
        v4 = v + v + v + v
        idx_ref[rj, pl.ds(k * 16, 16)] = jnp.where(
            v >= 3 * H4, v4 - 3 * FLAT_MINUS,
            jnp.where(v >= 2 * H4, v4 - 2 * FLAT_MINUS,
                      jnp.where(v >= H4, v4 - FLAT_MINUS, v4)))
    for j in range(KSUB):
      pltpu.async_copy(tbl_hbm.at[idx_ref.at[j]],
                       rows_ref.at[pl.ds(j * SUB, SUB)], sem)

  load_and_fire(0, *bufs[0])
  load_and_fire(1, *bufs[1])

  def pair_body(g, carry):
    for bi in range(2):
      idx_ref, rows_ref, sem = bufs[bi]
      chunk = g * 2 + bi
      pltpu.make_async_copy(tbl_hbm.at[pl.ds(0, CHUNK)], rows_ref, sem).wait()
      for j in range(KSUB):
        pltpu.sync_copy(rows_ref.at[pl.ds(j * SUB, SUB)],
                        acc_sh.at[seg2d.at[chunk * KSUB + j]], add=True)
      nxt = chunk + 2
      @pl.when(nxt < NCHUNK)
      def _():
        load_and_fire(nxt, idx_ref, rows_ref, sem)
    return carry
  lax.fori_loop(0, NCHUNK // 2, pair_body, 0)

  pltpu.sync_copy(acc_sh.at[pl.ds(sbase, ROWS_W)],
                  feat_hbm.at[pl.ds(w * ROWS_W, ROWS_W)])


_sc_pool = pl.kernel(
    _sc_pool_body,
    out_type=jax.ShapeDtypeStruct((B, D2), jnp.float32),
    mesh=plsc.VectorSubcoreMesh(core_axis_name="c", subcore_axis_name="s"),
    scratch_types=[
        pltpu.VMEM((KSUB, SUB), jnp.int32),
        pltpu.VMEM((KSUB, SUB), jnp.int32),
        pltpu.VMEM((CHUNK, D2), jnp.float32),
        pltpu.VMEM((CHUNK, D2), jnp.float32),
        pltpu.VMEM((NCHUNK * KSUB, SUB), jnp.int32),
        pltpu.VMEM_SHARED((NS * ROWS_W, D2), jnp.float32),
        pltpu.SemaphoreType.DMA,
        pltpu.SemaphoreType.DMA,
    ],
    name="sc_embedding_bag_pool",
    compiler_params=pltpu.CompilerParams(use_tc_tiling_on_sc=False),
)

V_BLK = 4096
CONV_GRID = H4 // V_BLK
CLAMP_BLK = VOCAB // V_BLK


def _conv_body(r0_ref, r1_ref, r2_ref, r3_ref, out_ref):
  eye = (lax.broadcasted_iota(jnp.int32, (4 * D2, 4 * D2), 0)
         == lax.broadcasted_iota(jnp.int32, (4 * D2, 4 * D2), 1)
         ).astype(jnp.float32)
  both = jnp.concatenate(
      [r0_ref[...], r1_ref[...], r2_ref[...], r3_ref[...]], axis=0)
  out_ref[...] = lax.dot_general(both, eye, (((0,), (0,)), ((), ())),
                                 preferred_element_type=jnp.float32)


def _make_conv(rb):
  def mk(off):
    return lambda i: (rb, jnp.minimum(i + off, CLAMP_BLK))
  return pl.pallas_call(
      _conv_body,
      grid=(CONV_GRID,),
      in_specs=[pl.BlockSpec((D2, V_BLK), mk(o))
                for o in (0, CONV_GRID, 2 * CONV_GRID, 3 * CONV_GRID)],
      out_specs=pl.BlockSpec((V_BLK, 4 * D2), lambda i: (i, 0)),
      out_shape=jax.ShapeDtypeStruct((H4, 4 * D2), jnp.float32),
  )


_convA = _make_conv(0)
_convB = _make_conv(1)

NPAD = 1024
BLK_B = 512


def _mm1_body(fa_ref, w_ref, o_ref):
  o_ref[...] = jnp.dot(
      fa_ref[...], w_ref[...], preferred_element_type=jnp.float32,
      precision=lax.Precision.HIGHEST) * jnp.float32(1.0 / L)


_mm1 = pl.pallas_call(
    _mm1_body,
    grid=(B // BLK_B,),
    in_specs=[
        pl.BlockSpec((BLK_B, D2), lambda i: (i, 0)),
        pl.BlockSpec((D2, NPAD), lambda i: (0, 0)),
    ],
    out_specs=pl.BlockSpec((BLK_B, NPAD), lambda i: (i, 0)),
    out_shape=jax.ShapeDtypeStruct((B, NPAD), jnp.float32),
)


def _mm2_body(fb_ref, w_ref, b_ref, p_ref, o_ref):
  o_ref[...] = (
      jnp.dot(fb_ref[...], w_ref[...], preferred_element_type=jnp.float32,
              precision=lax.Precision.HIGHEST) * jnp.float32(1.0 / L)
      + p_ref[...] + b_ref[...])


_mm2 = pl.pallas_call(
    _mm2_body,
    grid=(B // BLK_B,),
    in_specs=[
        pl.BlockSpec((BLK_B, D2), lambda i: (i, 0)),
        pl.BlockSpec((D2, NPAD), lambda i: (0, 0)),
        pl.BlockSpec((1, NPAD), lambda i: (0, 0)),
        pl.BlockSpec((BLK_B, NPAD), lambda i: (i, 0)),
    ],
    out_specs=pl.BlockSpec((BLK_B, NPAD), lambda i: (i, 0)),
    out_shape=jax.ShapeDtypeStruct((B, NPAD), jnp.float32),
)


def kernel(x, table, W, b):
  x2d = x.astype(jnp.int32).reshape(-1, SUB)
  tt = table.T
  tA = _convA(tt, tt, tt, tt).reshape(4 * H4, D2)
  fA = _sc_pool(x2d, tA)
  tB = _convB(tt, tt, tt, tt).reshape(4 * H4, D2)
  fB = _sc_pool(x2d, tB)
  Wp = jnp.pad(W, ((0, 0), (0, NPAD - NUM_CLASSES)))
  bp = jnp.pad(b, (0, NPAD - NUM_CLASSES)).reshape(1, NPAD)
  part = _mm1(fA, Wp[:D2])
  out = _mm2(fB, Wp[D2:], bp, part)
  return out[:, :NUM_CLASSES]


# speedup vs baseline: 1.0592x; 1.0007x over previous
import jax
import jax.numpy as jnp
from jax import lax
from jax.experimental import pallas as pl
from jax.experimental.pallas import tpu as pltpu
from jax.experimental.pallas import tpu_sc as plsc

B, L, D = 4096, 200, 64
VOCAB = 1000000
NUM_CLASSES = 1000
D2 = D // 2
H4 = 262144
FLAT_MINUS = 4 * H4 - 1

NC, NS = 2, 16
NW = NC * NS
ROWS_W = B // NW
TOK_W = ROWS_W * L
SUB = 128
KSUB = 5
CHUNK = SUB * KSUB
NCHUNK = TOK_W // CHUNK
XROWS_W = TOK_W // SUB


def _sc_pool_body(x_hbm, tbl_hbm, feat_hbm,
                  idx0, idx1, rows0, rows1, seg2d, acc_sh, sem0, sem1):
  c = lax.axis_index("c")
  s = lax.axis_index("s")
  w = c * NS + s
  xrow0 = w * XROWS_W
  sbase = s * ROWS_W

  iota = lax.iota(jnp.int32, 16)
  lvec = jnp.full((16,), L, jnp.int32)

  def seg_step(v, carry):
    t = iota + v * 16
    seg = lax.div(t, lvec) + sbase
    seg2d[v // 8, pl.ds((v % 8) * 16, 16)] = seg
    return carry
  lax.fori_loop(0, TOK_W // 16, seg_step, 0)

  zero16 = jnp.zeros((16,), jnp.float32)
  def z_step(i, carry):
    rows0[i // 2, pl.ds((i % 2) * 16, 16)] = zero16
    return carry
  lax.fori_loop(0, ROWS_W * (D2 // 16), z_step, 0)
  pltpu.sync_copy(rows0.at[pl.ds(0, ROWS_W)], acc_sh.at[pl.ds(sbase, ROWS_W)])

  bufs = ((idx0, rows0, sem0), (idx1, rows1, sem1))

  def load_and_fire(chunk, idx_ref, rows_ref, sem):
    pltpu.sync_copy(x_hbm.at[pl.ds(xrow0 + chunk * KSUB, KSUB)], idx_ref)
    for rj in range(KSUB):
      for k in range(SUB // 16):
        v = idx_ref[rj, pl.ds(k * 16, 16)]
        v4 = v + v + v + v
        idx_ref[rj, pl.ds(k * 16, 16)] = jnp.where(
            v >= 3 * H4, v4 - 3 * FLAT_MINUS,
            jnp.where(v >= 2 * H4, v4 - 2 * FLAT_MINUS,
                      jnp.where(v >= H4, v4 - FLAT_MINUS, v4)))
    for j in range(KSUB):
      pltpu.async_copy(tbl_hbm.at[idx_ref.at[j]],
                       rows_ref.at[pl.ds(j * SUB, SUB)], sem)

  load_and_fire(0, *bufs[0])
  load_and_fire(1, *bufs[1])

  def pair_body(g, carry):
    for bi in range(2):
      idx_ref, rows_ref, sem = bufs[bi]
      chunk = g * 2 + bi
      pltpu.make_async_copy(tbl_hbm.at[pl.ds(0, CHUNK)], rows_ref, sem).wait()
      for j in range(KSUB):
        pltpu.sync_copy(rows_ref.at[pl.ds(j * SUB, SUB)],
                        acc_sh.at[seg2d.at[chunk * KSUB + j]], add=True)
      nxt = chunk + 2
      @pl.when(nxt < NCHUNK)
      def _():
        load_and_fire(nxt, idx_ref, rows_ref, sem)
    return carry
  lax.fori_loop(0, NCHUNK // 2, pair_body, 0)

  pltpu.sync_copy(acc_sh.at[pl.ds(sbase, ROWS_W)],
                  feat_hbm.at[pl.ds(w * ROWS_W, ROWS_W)])


_sc_pool = pl.kernel(
    _sc_pool_body,
    out_type=jax.ShapeDtypeStruct((B, D2), jnp.float32),
    mesh=plsc.VectorSubcoreMesh(core_axis_name="c", subcore_axis_name="s"),
    scratch_types=[
        pltpu.VMEM((KSUB, SUB), jnp.int32),
        pltpu.VMEM((KSUB, SUB), jnp.int32),
        pltpu.VMEM((CHUNK, D2), jnp.float32),
        pltpu.VMEM((CHUNK, D2), jnp.float32),
        pltpu.VMEM((NCHUNK * KSUB, SUB), jnp.int32),
        pltpu.VMEM_SHARED((NS * ROWS_W, D2), jnp.float32),
        pltpu.SemaphoreType.DMA,
        pltpu.SemaphoreType.DMA,
    ],
    name="sc_embedding_bag_pool",
    compiler_params=pltpu.CompilerParams(use_tc_tiling_on_sc=False),
)

V_BLK = 4096
CONV_GRID = H4 // V_BLK
CLAMP_BLK = VOCAB // V_BLK


def _conv_body(r0_ref, r1_ref, r2_ref, r3_ref, out_ref):
  eye = (lax.broadcasted_iota(jnp.int32, (4 * D2, 4 * D2), 0)
         == lax.broadcasted_iota(jnp.int32, (4 * D2, 4 * D2), 1)
         ).astype(jnp.float32)
  both = jnp.concatenate(
      [r0_ref[...], r1_ref[...], r2_ref[...], r3_ref[...]], axis=0)
  out_ref[...] = lax.dot_general(both, eye, (((0,), (0,)), ((), ())),
                                 preferred_element_type=jnp.float32)


def _make_conv(rb):
  def mk(off):
    return lambda i: (rb, jnp.minimum(i + off, CLAMP_BLK))
  return pl.pallas_call(
      _conv_body,
      grid=(CONV_GRID,),
      in_specs=[pl.BlockSpec((D2, V_BLK), mk(o))
                for o in (0, CONV_GRID, 2 * CONV_GRID, 3 * CONV_GRID)],
      out_specs=pl.BlockSpec((V_BLK, 4 * D2), lambda i: (i, 0)),
      out_shape=jax.ShapeDtypeStruct((H4, 4 * D2), jnp.float32),
  )


_convA = _make_conv(0)
_convB = _make_conv(1)

NPAD = 1024
BLK_B = 512


def _mm1_body(fa_ref, w_ref, o_ref):
  o_ref[...] = jnp.dot(
      fa_ref[...], w_ref[...], preferred_element_type=jnp.float32,
      precision=lax.Precision.HIGHEST) * jnp.float32(1.0 / L)


_mm1 = pl.pallas_call(
    _mm1_body,
    grid=(B // BLK_B,),
    in_specs=[
        pl.BlockSpec((BLK_B, D2), lambda i: (i, 0)),
        pl.BlockSpec((D2, NPAD), lambda i: (0, 0)),
    ],
    out_specs=pl.BlockSpec((BLK_B, NPAD), lambda i: (i, 0)),
    out_shape=jax.ShapeDtypeStruct((B, NPAD), jnp.float32),
)


def _mm2_body(fb_ref, w_ref, b_ref, p_ref, o_ref):
  o_ref[...] = (
      jnp.dot(fb_ref[...], w_ref[...], preferred_element_type=jnp.float32,
              precision=lax.Precision.HIGHEST) * jnp.float32(1.0 / L)
      + p_ref[...] + b_ref[...])


_mm2 = pl.pallas_call(
    _mm2_body,
    grid=(B // BLK_B,),
    in_specs=[
        pl.BlockSpec((BLK_B, D2), lambda i: (i, 0)),
        pl.BlockSpec((D2, NPAD), lambda i: (0, 0)),
        pl.BlockSpec((1, NPAD), lambda i: (0, 0)),
        pl.BlockSpec((BLK_B, NPAD), lambda i: (i, 0)),
    ],
    out_specs=pl.BlockSpec((BLK_B, NPAD), lambda i: (i, 0)),
    out_shape=jax.ShapeDtypeStruct((B, NPAD), jnp.float32),
)


def kernel(x, table, W, b):
  x2d = x.astype(jnp.int32).reshape(-1, SUB)
  tt = table.T
  tA = _convA(tt, tt, tt, tt).reshape(4 * H4, D2)
  fA = _sc_pool(x2d, tA)
  tB = _convB(tt, tt, tt, tt).reshape(4 * H4, D2)
  Wp = jnp.pad(W, ((0, 0), (0, NPAD - NUM_CLASSES)))
  bp = jnp.pad(b, (0, NPAD - NUM_CLASSES)).reshape(1, NPAD)
  part = _mm1(fA, Wp[:D2])
  fB = _sc_pool(x2d, tB)
  out = _mm2(fB, Wp[D2:], bp, part)
  return out[:, :NUM_CLASSES]


# speedup vs baseline: 1.1226x; 1.0598x over previous
import jax
import jax.numpy as jnp
from jax import lax
from jax.experimental import pallas as pl
from jax.experimental.pallas import tpu as pltpu
from jax.experimental.pallas import tpu_sc as plsc

B, L, D = 4096, 200, 64
VOCAB = 1000000
NUM_CLASSES = 1000
D2 = D // 2
H4 = 262144
FLAT_MINUS = 4 * H4 - 1

NC, NS = 2, 16
NW = NC * NS
ROWS_W = B // NW
TOK_W = ROWS_W * L
SUB = 128
KSUB = 5
CHUNK = SUB * KSUB
NCHUNK = TOK_W // CHUNK
XROWS_W = TOK_W // SUB


def _sc_pool_body(x_hbm, tbl_hbm, feat_hbm,
                  idx0, idx1, rows0, rows1, seg2d, acc_sh, sem0, sem1):
  c = lax.axis_index("c")
  s = lax.axis_index("s")
  w = c * NS + s
  xrow0 = w * XROWS_W
  sbase = s * ROWS_W

  iota = lax.iota(jnp.int32, 16)
  lvec = jnp.full((16,), L, jnp.int32)

  def seg_step(v, carry):
    t = iota + v * 16
    seg = lax.div(t, lvec) + sbase
    seg2d[v // 8, pl.ds((v % 8) * 16, 16)] = seg
    return carry
  lax.fori_loop(0, TOK_W // 16, seg_step, 0)

  zero16 = jnp.zeros((16,), jnp.float32)
  def z_step(i, carry):
    rows0[i // 2, pl.ds((i % 2) * 16, 16)] = zero16
    return carry
  lax.fori_loop(0, ROWS_W * (D2 // 16), z_step, 0)
  pltpu.sync_copy(rows0.at[pl.ds(0, ROWS_W)], acc_sh.at[pl.ds(sbase, ROWS_W)])

  bufs = ((idx0, rows0, sem0), (idx1, rows1, sem1))

  def load_and_fire(chunk, idx_ref, rows_ref, sem):
    pltpu.sync_copy(x_hbm.at[pl.ds(xrow0 + chunk * KSUB, KSUB)], idx_ref)
    for rj in range(KSUB):
      for k in range(SUB // 16):
        v = idx_ref[rj, pl.ds(k * 16, 16)]
        v4 = v + v + v + v
        idx_ref[rj, pl.ds(k * 16, 16)] = jnp.where(
            v >= 3 * H4, v4 - 3 * FLAT_MINUS,
            jnp.where(v >= 2 * H4, v4 - 2 * FLAT_MINUS,
                      jnp.where(v >= H4, v4 - FLAT_MINUS, v4)))
    for j in range(KSUB):
      pltpu.async_copy(tbl_hbm.at[idx_ref.at[j]],
                       rows_ref.at[pl.ds(j * SUB, SUB)], sem)

  load_and_fire(0, *bufs[0])
  load_and_fire(1, *bufs[1])

  def pair_body(g, carry):
    for bi in range(2):
      idx_ref, rows_ref, sem = bufs[bi]
      chunk = g * 2 + bi
      pltpu.make_async_copy(tbl_hbm.at[pl.ds(0, CHUNK)], rows_ref, sem).wait()
      for j in range(KSUB):
        pltpu.sync_copy(rows_ref.at[pl.ds(j * SUB, SUB)],
                        acc_sh.at[seg2d.at[chunk * KSUB + j]], add=True)
      nxt = chunk + 2
      @pl.when(nxt < NCHUNK)
      def _():
        load_and_fire(nxt, idx_ref, rows_ref, sem)
    return carry
  lax.fori_loop(0, NCHUNK // 2, pair_body, 0)

  pltpu.sync_copy(acc_sh.at[pl.ds(sbase, ROWS_W)],
                  feat_hbm.at[pl.ds(w * ROWS_W, ROWS_W)])


_sc_pool = pl.kernel(
    _sc_pool_body,
    out_type=jax.ShapeDtypeStruct((B, D2), jnp.float32),
    mesh=plsc.VectorSubcoreMesh(core_axis_name="c", subcore_axis_name="s"),
    scratch_types=[
        pltpu.VMEM((KSUB, SUB), jnp.int32),
        pltpu.VMEM((KSUB, SUB), jnp.int32),
        pltpu.VMEM((CHUNK, D2), jnp.float32),
        pltpu.VMEM((CHUNK, D2), jnp.float32),
        pltpu.VMEM((NCHUNK * KSUB, SUB), jnp.int32),
        pltpu.VMEM_SHARED((NS * ROWS_W, D2), jnp.float32),
        pltpu.SemaphoreType.DMA,
        pltpu.SemaphoreType.DMA,
    ],
    name="sc_embedding_bag_pool",
    compiler_params=pltpu.CompilerParams(use_tc_tiling_on_sc=False),
)

V_BLK = 8192
CONV_GRID = H4 // V_BLK
CLAMP_BLK = VOCAB // V_BLK


def _conv_body(r0_ref, r1_ref, r2_ref, r3_ref, out_ref):
  eye = (lax.broadcasted_iota(jnp.int32, (4 * D2, 4 * D2), 0)
         == lax.broadcasted_iota(jnp.int32, (4 * D2, 4 * D2), 1)
         ).astype(jnp.float32)
  both = jnp.concatenate(
      [r0_ref[...], r1_ref[...], r2_ref[...], r3_ref[...]], axis=0)
  out_ref[...] = lax.dot_general(both, eye, (((0,), (0,)), ((), ())),
                                 preferred_element_type=jnp.float32)


def _make_conv(rb):
  def mk(off):
    return lambda i: (rb, jnp.minimum(i + off, CLAMP_BLK))
  return pl.pallas_call(
      _conv_body,
      grid=(CONV_GRID,),
      in_specs=[pl.BlockSpec((D2, V_BLK), mk(o))
                for o in (0, CONV_GRID, 2 * CONV_GRID, 3 * CONV_GRID)],
      out_specs=pl.BlockSpec((V_BLK, 4 * D2), lambda i: (i, 0)),
      out_shape=jax.ShapeDtypeStruct((H4, 4 * D2), jnp.float32),
  )


_convA = _make_conv(0)
_convB = _make_conv(1)

NPAD = 1024
BLK_B = 512


def _mm_body(fa_ref, fb_ref, w_ref, b_ref, o_ref):
  f = jnp.concatenate([fa_ref[...], fb_ref[...]], axis=1)
  o_ref[...] = (
      jnp.dot(f, w_ref[...], preferred_element_type=jnp.float32,
              precision=lax.Precision.HIGHEST) * jnp.float32(1.0 / L)
      + b_ref[...])


_mm = pl.pallas_call(
    _mm_body,
    grid=(B // BLK_B,),
    in_specs=[
        pl.BlockSpec((BLK_B, D2), lambda i: (i, 0)),
        pl.BlockSpec((BLK_B, D2), lambda i: (i, 0)),
        pl.BlockSpec((D, NPAD), lambda i: (0, 0)),
        pl.BlockSpec((1, NPAD), lambda i: (0, 0)),
    ],
    out_specs=pl.BlockSpec((BLK_B, NPAD), lambda i: (i, 0)),
    out_shape=jax.ShapeDtypeStruct((B, NPAD), jnp.float32),
)


def kernel(x, table, W, b):
  x2d = x.astype(jnp.int32).reshape(-1, SUB)
  tt = table.T
  tA = _convA(tt, tt, tt, tt).reshape(4 * H4, D2)
  fA = _sc_pool(x2d, tA)
  tB = _convB(tt, tt, tt, tt).reshape(4 * H4, D2)
  fB = _sc_pool(x2d, tB)
  Wp = jnp.pad(W, ((0, 0), (0, NPAD - NUM_CLASSES)))
  bp = jnp.pad(b, (0, NPAD - NUM_CLASSES)).reshape(1, NPAD)
  out = _mm(fA, fB, Wp, bp)
  return out[:, :NUM_CLASSES]
